# Initial kernel scaffold; baseline (speedup 1.0000x reference)
import jax
import jax.numpy as jnp
from jax.experimental import pallas as pl


def kernel(out, semantic_labels, group_labels):
    raise NotImplementedError("write your pallas kernel here")


# speedup vs baseline: 7.1463x; 7.1463x over previous
import functools
import jax
import jax.numpy as jnp
from jax import lax
from jax.experimental import pallas as pl
from jax.experimental.pallas import tpu as pltpu
from jax.experimental.pallas import tpu_sc as plsc

N = 32768
D = 64
W = 80
NSEG = 320
NC = 2
NS = 16
NW = NC * NS
CHUNK = N // NW
GROUPS = CHUNK // 16

_mesh = plsc.VectorSubcoreMesh(
    core_axis_name="c", subcore_axis_name="s", num_cores=NC, num_subcores=NS)


@functools.partial(
    pl.kernel,
    out_type=(
        jax.ShapeDtypeStruct((NC * NSEG, W), jnp.float32),
        jax.ShapeDtypeStruct((N // 128, 128), jnp.int32),
    ),
    mesh=_mesh,
    scratch_types=[
        pltpu.VMEM((CHUNK // 2, W), jnp.float32),
        pltpu.VMEM((CHUNK,), jnp.int32),
        pltpu.VMEM((CHUNK,), jnp.int32),
        pltpu.VMEM((CHUNK,), jnp.int32),
        pltpu.VMEM((8, 128), jnp.int32),
        pltpu.VMEM_SHARED((NSEG, W), jnp.float32),
        pltpu.SemaphoreType.DMA,
    ],
)
def _seg_sums(xpad, bidx, slab, clab, zer, partial, seg_out,
              xbuf, bbuf, sbuf, cbuf, seg2, shared, sem):
    cid = lax.axis_index("c")
    sid = lax.axis_index("s")
    wid = sid * NC + cid
    base = wid * CHUNK
    pltpu.sync_copy(bidx.at[pl.ds(base, CHUNK)], bbuf)
    pltpu.sync_copy(slab.at[pl.ds(base, CHUNK)], sbuf)
    pltpu.sync_copy(clab.at[pl.ds(base, CHUNK)], cbuf)
    @pl.when(sid < 10)
    def _zero():
        pltpu.sync_copy(zer.at[pl.ds(sid * 32, 32)],
                        shared.at[pl.ds(sid * 32, 32)])
    for i in range(GROUPS):
        b = bbuf[pl.ds(i * 16, 16)]
        s = sbuf[pl.ds(i * 16, 16)]
        g = cbuf[pl.ds(i * 16, 16)]
        seg2[i // 8, pl.ds((i % 8) * 16, 16)] = b * 80 + s * 20 + g
    pltpu.sync_copy(seg2, seg_out.at[pl.ds(wid * 8, 8)])
    plsc.subcore_barrier()
    for h in range(2):
        pltpu.sync_copy(xpad.at[pl.ds(base + h * (CHUNK // 2), CHUNK // 2)],
                        xbuf)
        descs = [pltpu.async_copy(xbuf.at[pl.ds(k * 128, 128)],
                                  shared.at[seg2.at[h * 4 + k]], sem,
                                  add=True)
                 for k in range(4)]
        for dsc in descs:
            dsc.wait()
    plsc.subcore_barrier()

    @pl.when(sid < 10)
    def _writeout():
        pltpu.sync_copy(shared.at[pl.ds(sid * 32, 32)],
                        partial.at[pl.ds(cid * NSEG + sid * 32, 32)])


def _combine_body(partial_ref, table_ref, svec_ref):
    p = partial_ref[...]
    sums = p[0:NSEG] + p[NSEG:2 * NSEG]
    cnt = sums[:, 64:65]
    safe_cnt = jnp.maximum(cnt, 1.0)
    means80 = sums / safe_cnt
    m64 = means80[:, 0:64]
    present = (cnt > 0.0).astype(jnp.float32)

    segr = lax.broadcasted_iota(jnp.int32, (NSEG, 16), 0)
    comc = lax.broadcasted_iota(jnp.int32, (NSEG, 16), 1)
    Mc = (segr // 20 == comc).astype(jnp.float32)
    segc = lax.broadcasted_iota(jnp.int32, (16, NSEG), 1)
    comr = lax.broadcasted_iota(jnp.int32, (16, NSEG), 0)
    McT = (segc // 20 == comr).astype(jnp.float32)

    dn = (((1,), (0,)), ((), ()))
    hi = jax.lax.Precision.HIGHEST
    dot = functools.partial(lax.dot_general, dimension_numbers=dn,
                            precision=hi, preferred_element_type=jnp.float32)
    n_combo = dot(McT, present)
    safe_n = jnp.maximum(n_combo, 1.0)
    cnt_combo = dot(McT, cnt)
    anyp = cnt_combo > 0.0
    n_seg = dot(Mc, n_combo)
    wn = jnp.where(present > 0.0,
                   1.0 / (safe_cnt * jnp.maximum(n_seg, 1.0)), 0.0)

    m2 = m64 * m64
    sqn = jnp.sum(m2, axis=1, keepdims=True)
    sqn_row = lax.dot_general(jnp.ones((1, 64), jnp.float32), m2,
                              dimension_numbers=(((1,), (1,)), ((), ())),
                              precision=hi,
                              preferred_element_type=jnp.float32)
    regseg = jnp.sqrt(sqn + 1e-12) * present
    reg_combo = dot(McT, regseg) / safe_n

    G = lax.dot_general(m64, m64,
                        dimension_numbers=(((1,), (1,)), ((), ())),
                        precision=hi,
                        preferred_element_type=jnp.float32)
    pair_sq = jnp.maximum(sqn + sqn_row - 2.0 * G, 0.0)
    d = jnp.sqrt(pair_sq + 1e-12)
    h = jnp.maximum(3.0 - d, 0.0) ** 2
    ri = lax.broadcasted_iota(jnp.int32, (NSEG, NSEG), 0)
    ci = lax.broadcasted_iota(jnp.int32, (NSEG, NSEG), 1)
    same = (ri // 20) == (ci // 20)
    offdiag = ri != ci
    pres_row = lax.dot_general(jnp.ones((1, 1), jnp.float32), present,
                               dimension_numbers=(((1,), (1,)), ((), ())),
                               precision=hi,
                               preferred_element_type=jnp.float32)
    prespair = (present > 0.0) & (pres_row > 0.0)
    hm = jnp.where(same & offdiag & prespair, h, 0.0)
    dist_row = jnp.sum(hm, axis=1, keepdims=True)
    dist_combo = dot(McT, dist_row)
    denom = jnp.maximum((n_combo - 1.0) * n_combo, 1.0)
    dist_loss = jnp.where(n_combo < 2.0, 0.0, dist_combo / denom)
    distreg = jnp.sum(jnp.where(anyp, dist_loss + 0.001 * reg_combo, 0.0))

    batc = lax.broadcasted_iota(jnp.int32, (4, NSEG), 1)
    batr = lax.broadcasted_iota(jnp.int32, (4, NSEG), 0)
    MbT = (batc // 80 == batr).astype(jnp.float32)
    nb = dot(MbT, cnt)
    nbatch = jnp.sum((nb > 0.0).astype(jnp.float32))

    table_ref[...] = jnp.concatenate(
        [m64, wn, jnp.zeros((NSEG, W - 65), jnp.float32)], axis=1)
    svec_ref[...] = jnp.concatenate(
        [jnp.reshape(distreg, (1, 1)), jnp.reshape(nbatch, (1, 1)),
         jnp.zeros((1, 126), jnp.float32)], axis=1)


def _combine(partial):
    return pl.pallas_call(
        _combine_body,
        out_shape=(
            jax.ShapeDtypeStruct((NSEG, W), jnp.float32),
            jax.ShapeDtypeStruct((1, 128), jnp.float32),
        ),
    )(partial)


@functools.partial(
    pl.kernel,
    out_type=jax.ShapeDtypeStruct((NW * 16,), jnp.float32),
    mesh=_mesh,
    compiler_params=pltpu.CompilerParams(needs_layout_passes=False),
    scratch_types=[
        pltpu.VMEM((D, CHUNK // 2), jnp.float32),
        pltpu.VMEM((NSEG * W,), jnp.float32),
        pltpu.VMEM((8, 128), jnp.int32),
        pltpu.VMEM((16,), jnp.float32),
        pltpu.SemaphoreType.DMA,
    ],
)
def _var_points(xT, seg_in, table, cpart, xtbuf, tbuf, seg2, obuf, sem):
    cid = lax.axis_index("c")
    sid = lax.axis_index("s")
    wid = sid * NC + cid
    d1 = pltpu.async_copy(table, tbuf, sem)
    d2 = pltpu.async_copy(seg_in.at[pl.ds(wid * 8, 8)], seg2, sem)
    d1.wait()
    d2.wait()

    def inner(h, r, c, acc):
        gbase = r * 128 + c * 16
        segv = seg2[h * 4 + r, pl.ds(c * 16, 16)]
        fidx0 = segv * W
        saccs = [jnp.zeros((16,), jnp.float32) for _ in range(4)]
        for j in range(D):
            xj = xtbuf[j, pl.ds(gbase, 16)]
            mj = plsc.load_gather(tbuf, [fidx0 + j])
            dv = xj - mj
            saccs[j % 4] = saccs[j % 4] + dv * dv
        sacc = (saccs[0] + saccs[1]) + (saccs[2] + saccs[3])
        wv = plsc.load_gather(tbuf, [fidx0 + D])
        s = sacc + 1e-12
        si = plsc.bitcast(s, jnp.int32)
        y = plsc.bitcast(lax.shift_right_logical(si, 1) + 0x1FBD1DF5,
                         jnp.float32)
        for _ in range(3):
            y = 0.5 * (y + s / y)
        hv = jnp.maximum(y - 0.5, 0.0)
        return acc + hv * hv * wv

    varacc = jnp.zeros((16,), jnp.float32)
    for h in range(2):
        pltpu.sync_copy(xT.at[wid * 2 + h], xtbuf)

        def outer(r, acc, h=h):
            return lax.fori_loop(0, 8, lambda c, a: inner(h, r, c, a), acc)

        varacc = lax.fori_loop(0, 4, outer, varacc)
    obuf[...] = varacc
    pltpu.sync_copy(obuf, cpart.at[pl.ds(wid * 16, 16)])


def _finish_body(cpart_ref, svec_ref, out_ref):
    sv = svec_ref[...]
    total = (jnp.sum(cpart_ref[...]) + sv[0, 0]) / sv[0, 1]
    out_ref[...] = jnp.reshape(total, (1, 1))


def _finish(cpart, svec):
    return pl.pallas_call(
        _finish_body,
        out_shape=jax.ShapeDtypeStruct((1, 1), jnp.float32),
    )(cpart, svec)


@jax.jit
def kernel(out, semantic_labels, group_labels):
    x = out.astype(jnp.float32)
    xpad = jnp.concatenate(
        [x, jnp.ones((N, 1), jnp.float32), jnp.zeros((N, W - D - 1), jnp.float32)],
        axis=1)
    bidx = semantic_labels[:, 3].astype(jnp.int32)
    slab = semantic_labels[:, 4].astype(jnp.int32)
    clab = group_labels[:, 4].astype(jnp.int32)
    zer = jnp.zeros((NSEG, W), jnp.float32)
    xT = x.T.reshape(D, NW * 2, CHUNK // 2).transpose(1, 0, 2)

    partial, seg = _seg_sums(xpad, bidx, slab, clab, zer)
    table, svec = _combine(partial)
    cpart = _var_points(xT, seg, table.reshape(NSEG * W))
    res = _finish(cpart.reshape(4, 128), svec)
    return res[0, 0]


# speedup vs baseline: 8.2538x; 1.1550x over previous
import functools
import jax
import jax.numpy as jnp
from jax import lax
from jax.experimental import pallas as pl
from jax.experimental.pallas import tpu as pltpu
from jax.experimental.pallas import tpu_sc as plsc

N = 32768
D = 64
W = 80
NSEG = 320
NC = 2
NS = 16
NW = NC * NS
CHUNK = N // NW
GROUPS = CHUNK // 16

_mesh = plsc.VectorSubcoreMesh(
    core_axis_name="c", subcore_axis_name="s", num_cores=NC, num_subcores=NS)


@functools.partial(
    pl.kernel,
    out_type=(
        jax.ShapeDtypeStruct((NC * NSEG, W), jnp.float32),
        jax.ShapeDtypeStruct((N // 128, 128), jnp.int32),
    ),
    mesh=_mesh,
    scratch_types=[
        pltpu.VMEM((CHUNK // 2, W), jnp.float32),
        pltpu.VMEM((CHUNK,), jnp.int32),
        pltpu.VMEM((CHUNK,), jnp.int32),
        pltpu.VMEM((CHUNK,), jnp.int32),
        pltpu.VMEM((8, 128), jnp.int32),
        pltpu.VMEM_SHARED((NSEG, W), jnp.float32),
    ],
)
def _seg_sums(xpad, bidx, slab, clab, zer, partial, seg_out,
              xbuf, bbuf, sbuf, cbuf, seg2, shared):
    cid = lax.axis_index("c")
    sid = lax.axis_index("s")
    wid = sid * NC + cid
    base = wid * CHUNK
    pltpu.sync_copy(bidx.at[pl.ds(base, CHUNK)], bbuf)
    pltpu.sync_copy(slab.at[pl.ds(base, CHUNK)], sbuf)
    pltpu.sync_copy(clab.at[pl.ds(base, CHUNK)], cbuf)
    @pl.when(sid < 10)
    def _zero():
        pltpu.sync_copy(zer.at[pl.ds(sid * 32, 32)],
                        shared.at[pl.ds(sid * 32, 32)])
    for i in range(GROUPS):
        b = bbuf[pl.ds(i * 16, 16)]
        s = sbuf[pl.ds(i * 16, 16)]
        g = cbuf[pl.ds(i * 16, 16)]
        seg2[i // 8, pl.ds((i % 8) * 16, 16)] = b * 80 + s * 20 + g
    pltpu.sync_copy(seg2, seg_out.at[pl.ds(wid * 8, 8)])
    plsc.subcore_barrier()
    for h in range(2):
        pltpu.sync_copy(xpad.at[pl.ds(base + h * (CHUNK // 2), CHUNK // 2)],
                        xbuf)
        for k in range(4):
            pltpu.sync_copy(xbuf.at[pl.ds(k * 128, 128)],
                            shared.at[seg2.at[h * 4 + k]], add=True)
    plsc.subcore_barrier()

    @pl.when(sid < 10)
    def _writeout():
        pltpu.sync_copy(shared.at[pl.ds(sid * 32, 32)],
                        partial.at[pl.ds(cid * NSEG + sid * 32, 32)])


def _combine_body(partial_ref, table_ref, svec_ref):
    p = partial_ref[...]
    sums = p[0:NSEG] + p[NSEG:2 * NSEG]
    cnt = sums[:, 64:65]
    safe_cnt = jnp.maximum(cnt, 1.0)
    means80 = sums / safe_cnt
    m64 = means80[:, 0:64]
    present = (cnt > 0.0).astype(jnp.float32)

    segr = lax.broadcasted_iota(jnp.int32, (NSEG, 16), 0)
    comc = lax.broadcasted_iota(jnp.int32, (NSEG, 16), 1)
    Mc = (segr // 20 == comc).astype(jnp.float32)
    segc = lax.broadcasted_iota(jnp.int32, (16, NSEG), 1)
    comr = lax.broadcasted_iota(jnp.int32, (16, NSEG), 0)
    McT = (segc // 20 == comr).astype(jnp.float32)

    dn = (((1,), (0,)), ((), ()))
    hi = jax.lax.Precision.HIGHEST
    dot = functools.partial(lax.dot_general, dimension_numbers=dn,
                            precision=hi, preferred_element_type=jnp.float32)
    n_combo = dot(McT, present)
    safe_n = jnp.maximum(n_combo, 1.0)
    cnt_combo = dot(McT, cnt)
    anyp = cnt_combo > 0.0
    n_seg = dot(Mc, n_combo)
    wn = jnp.where(present > 0.0,
                   1.0 / (safe_cnt * jnp.maximum(n_seg, 1.0)), 0.0)

    m2 = m64 * m64
    sqn = jnp.sum(m2, axis=1, keepdims=True)
    sqn_row = lax.dot_general(jnp.ones((1, 64), jnp.float32), m2,
                              dimension_numbers=(((1,), (1,)), ((), ())),
                              precision=hi,
                              preferred_element_type=jnp.float32)
    regseg = jnp.sqrt(sqn + 1e-12) * present
    reg_combo = dot(McT, regseg) / safe_n

    G = lax.dot_general(m64, m64,
                        dimension_numbers=(((1,), (1,)), ((), ())),
                        precision=hi,
                        preferred_element_type=jnp.float32)
    pair_sq = jnp.maximum(sqn + sqn_row - 2.0 * G, 0.0)
    d = jnp.sqrt(pair_sq + 1e-12)
    h = jnp.maximum(3.0 - d, 0.0) ** 2
    ri = lax.broadcasted_iota(jnp.int32, (NSEG, NSEG), 0)
    ci = lax.broadcasted_iota(jnp.int32, (NSEG, NSEG), 1)
    same = (ri // 20) == (ci // 20)
    offdiag = ri != ci
    pres_row = lax.dot_general(jnp.ones((1, 1), jnp.float32), present,
                               dimension_numbers=(((1,), (1,)), ((), ())),
                               precision=hi,
                               preferred_element_type=jnp.float32)
    prespair = (present > 0.0) & (pres_row > 0.0)
    hm = jnp.where(same & offdiag & prespair, h, 0.0)
    dist_row = jnp.sum(hm, axis=1, keepdims=True)
    dist_combo = dot(McT, dist_row)
    denom = jnp.maximum((n_combo - 1.0) * n_combo, 1.0)
    dist_loss = jnp.where(n_combo < 2.0, 0.0, dist_combo / denom)
    distreg = jnp.sum(jnp.where(anyp, dist_loss + 0.001 * reg_combo, 0.0))

    batc = lax.broadcasted_iota(jnp.int32, (4, NSEG), 1)
    batr = lax.broadcasted_iota(jnp.int32, (4, NSEG), 0)
    MbT = (batc // 80 == batr).astype(jnp.float32)
    nb = dot(MbT, cnt)
    nbatch = jnp.sum((nb > 0.0).astype(jnp.float32))

    table_ref[...] = jnp.concatenate(
        [m64, wn, jnp.zeros((NSEG, W - 65), jnp.float32)], axis=1)
    svec_ref[...] = jnp.concatenate(
        [jnp.reshape(distreg, (1, 1)), jnp.reshape(nbatch, (1, 1)),
         jnp.zeros((1, 126), jnp.float32)], axis=1)


def _combine(partial):
    return pl.pallas_call(
        _combine_body,
        out_shape=(
            jax.ShapeDtypeStruct((NSEG, W), jnp.float32),
            jax.ShapeDtypeStruct((1, 128), jnp.float32),
        ),
    )(partial)


@functools.partial(
    pl.kernel,
    out_type=jax.ShapeDtypeStruct((NW * 16,), jnp.float32),
    mesh=_mesh,
    compiler_params=pltpu.CompilerParams(needs_layout_passes=False),
    scratch_types=[
        pltpu.VMEM((D * (CHUNK // 2) // 128, 128), jnp.float32),
        pltpu.VMEM((NSEG * W,), jnp.float32),
        pltpu.VMEM((8, 128), jnp.int32),
        pltpu.VMEM((16,), jnp.float32),
    ],
)
def _var_points(xT, seg_in, table, cpart, xtbuf, tbuf, seg2, obuf):
    cid = lax.axis_index("c")
    sid = lax.axis_index("s")
    wid = sid * NC + cid
    pltpu.sync_copy(table, tbuf)
    pltpu.sync_copy(seg_in.at[pl.ds(wid * 8, 8)], seg2)

    def inner(h, r, c, acc):
        segv = seg2[h * 4 + r, pl.ds(c * 16, 16)]
        fidx = segv * W
        sacc = jnp.zeros((16,), jnp.float32)
        for j in range(D):
            xj = xtbuf[j * 4 + r, pl.ds(c * 16, 16)]
            mj = plsc.load_gather(tbuf, [fidx])
            dv = xj - mj
            sacc = sacc + dv * dv
            fidx = fidx + 1
        wv = plsc.load_gather(tbuf, [fidx])
        s = sacc + 1e-12
        si = plsc.bitcast(s, jnp.int32)
        y = plsc.bitcast(lax.shift_right_logical(si, 1) + 0x1FBD1DF5,
                         jnp.float32)
        for _ in range(3):
            y = 0.5 * (y + s / y)
        hv = jnp.maximum(y - 0.5, 0.0)
        return acc + hv * hv * wv

    nrow = D * (CHUNK // 2) // 128
    varacc = jnp.zeros((16,), jnp.float32)
    for h in range(2):
        pltpu.sync_copy(xT.at[pl.ds((wid * 2 + h) * nrow, nrow)], xtbuf)

        def outer(r, acc, h=h):
            return lax.fori_loop(0, 8, lambda c, a: inner(h, r, c, a), acc)

        varacc = lax.fori_loop(0, 4, outer, varacc)
    obuf[...] = varacc
    pltpu.sync_copy(obuf, cpart.at[pl.ds(wid * 16, 16)])


def _finish_body(cpart_ref, svec_ref, out_ref):
    sv = svec_ref[...]
    total = (jnp.sum(cpart_ref[...]) + sv[0, 0]) / sv[0, 1]
    out_ref[...] = jnp.reshape(total, (1, 1))


def _finish(cpart, svec):
    return pl.pallas_call(
        _finish_body,
        out_shape=jax.ShapeDtypeStruct((1, 1), jnp.float32),
    )(cpart, svec)


@jax.jit
def kernel(out, semantic_labels, group_labels):
    x = out.astype(jnp.float32)
    xpad = jnp.concatenate(
        [x, jnp.ones((N, 1), jnp.float32), jnp.zeros((N, W - D - 1), jnp.float32)],
        axis=1)
    bidx = semantic_labels[:, 3].astype(jnp.int32)
    slab = semantic_labels[:, 4].astype(jnp.int32)
    clab = group_labels[:, 4].astype(jnp.int32)
    zer = jnp.zeros((NSEG, W), jnp.float32)
    xT = x.T.reshape(D, NW * 2, CHUNK // 2).transpose(1, 0, 2).reshape(
        NW * 2 * D * (CHUNK // 2) // 128, 128)

    partial, seg = _seg_sums(xpad, bidx, slab, clab, zer)
    table, svec = _combine(partial)
    cpart = _var_points(xT, seg, table.reshape(NSEG * W))
    res = _finish(cpart.reshape(4, 128), svec)
    return res[0, 0]


# speedup vs baseline: 8.2626x; 1.0011x over previous
import functools
import jax
import jax.numpy as jnp
from jax import lax
from jax.experimental import pallas as pl
from jax.experimental.pallas import tpu as pltpu
from jax.experimental.pallas import tpu_sc as plsc

N = 32768
D = 64
W = 80
NSEG = 320
NC = 2
NS = 16
NW = NC * NS
CHUNK = N // NW
GROUPS = CHUNK // 16

_mesh = plsc.VectorSubcoreMesh(
    core_axis_name="c", subcore_axis_name="s", num_cores=NC, num_subcores=NS)


@functools.partial(
    pl.kernel,
    out_type=(
        jax.ShapeDtypeStruct((NC * NSEG, W), jnp.float32),
        jax.ShapeDtypeStruct((N // 128, 128), jnp.int32),
    ),
    mesh=_mesh,
    scratch_types=[
        pltpu.VMEM((CHUNK // 2, W), jnp.float32),
        pltpu.VMEM((CHUNK,), jnp.int32),
        pltpu.VMEM((CHUNK,), jnp.int32),
        pltpu.VMEM((CHUNK,), jnp.int32),
        pltpu.VMEM((8, 128), jnp.int32),
        pltpu.VMEM_SHARED((NSEG, W), jnp.float32),
        pltpu.SemaphoreType.DMA,
    ],
)
def _seg_sums(xpad, bidx, slab, clab, zer, partial, seg_out,
              xbuf, bbuf, sbuf, cbuf, seg2, shared, sem):
    cid = lax.axis_index("c")
    sid = lax.axis_index("s")
    wid = sid * NC + cid
    base = wid * CHUNK
    pltpu.sync_copy(bidx.at[pl.ds(base, CHUNK)], bbuf)
    pltpu.sync_copy(slab.at[pl.ds(base, CHUNK)], sbuf)
    pltpu.sync_copy(clab.at[pl.ds(base, CHUNK)], cbuf)
    @pl.when(sid < 10)
    def _zero():
        pltpu.sync_copy(zer.at[pl.ds(sid * 32, 32)],
                        shared.at[pl.ds(sid * 32, 32)])
    for i in range(GROUPS):
        b = bbuf[pl.ds(i * 16, 16)]
        s = sbuf[pl.ds(i * 16, 16)]
        g = cbuf[pl.ds(i * 16, 16)]
        seg2[i // 8, pl.ds((i % 8) * 16, 16)] = b * 80 + s * 20 + g
    pltpu.sync_copy(seg2, seg_out.at[pl.ds(wid * 8, 8)])
    plsc.subcore_barrier()
    for h in range(2):
        pltpu.sync_copy(xpad.at[pl.ds(base + h * (CHUNK // 2), CHUNK // 2)],
                        xbuf)
        descs = [pltpu.async_copy(xbuf.at[pl.ds(k * 128, 128)],
                                  shared.at[seg2.at[h * 4 + k]], sem,
                                  add=True)
                 for k in range(4)]
        for dsc in descs:
            dsc.wait()
    plsc.subcore_barrier()

    @pl.when(sid < 10)
    def _writeout():
        pltpu.sync_copy(shared.at[pl.ds(sid * 32, 32)],
                        partial.at[pl.ds(cid * NSEG + sid * 32, 32)])


def _combine_body(partial_ref, table_ref, svec_ref):
    p = partial_ref[...]
    sums = p[0:NSEG] + p[NSEG:2 * NSEG]
    cnt = sums[:, 64:65]
    safe_cnt = jnp.maximum(cnt, 1.0)
    means80 = sums / safe_cnt
    m64 = means80[:, 0:64]
    present = (cnt > 0.0).astype(jnp.float32)

    segr = lax.broadcasted_iota(jnp.int32, (NSEG, 16), 0)
    comc = lax.broadcasted_iota(jnp.int32, (NSEG, 16), 1)
    Mc = (segr // 20 == comc).astype(jnp.float32)
    segc = lax.broadcasted_iota(jnp.int32, (16, NSEG), 1)
    comr = lax.broadcasted_iota(jnp.int32, (16, NSEG), 0)
    McT = (segc // 20 == comr).astype(jnp.float32)

    dn = (((1,), (0,)), ((), ()))
    hi = jax.lax.Precision.HIGHEST
    dot = functools.partial(lax.dot_general, dimension_numbers=dn,
                            precision=hi, preferred_element_type=jnp.float32)
    n_combo = dot(McT, present)
    safe_n = jnp.maximum(n_combo, 1.0)
    cnt_combo = dot(McT, cnt)
    anyp = cnt_combo > 0.0
    n_seg = dot(Mc, n_combo)
    wn = jnp.where(present > 0.0,
                   1.0 / (safe_cnt * jnp.maximum(n_seg, 1.0)), 0.0)

    m2 = m64 * m64
    sqn = jnp.sum(m2, axis=1, keepdims=True)
    sqn_row = lax.dot_general(jnp.ones((1, 64), jnp.float32), m2,
                              dimension_numbers=(((1,), (1,)), ((), ())),
                              precision=hi,
                              preferred_element_type=jnp.float32)
    regseg = jnp.sqrt(sqn + 1e-12) * present
    reg_combo = dot(McT, regseg) / safe_n

    G = lax.dot_general(m64, m64,
                        dimension_numbers=(((1,), (1,)), ((), ())),
                        precision=hi,
                        preferred_element_type=jnp.float32)
    pair_sq = jnp.maximum(sqn + sqn_row - 2.0 * G, 0.0)
    d = jnp.sqrt(pair_sq + 1e-12)
    h = jnp.maximum(3.0 - d, 0.0) ** 2
    ri = lax.broadcasted_iota(jnp.int32, (NSEG, NSEG), 0)
    ci = lax.broadcasted_iota(jnp.int32, (NSEG, NSEG), 1)
    same = (ri // 20) == (ci // 20)
    offdiag = ri != ci
    pres_row = lax.dot_general(jnp.ones((1, 1), jnp.float32), present,
                               dimension_numbers=(((1,), (1,)), ((), ())),
                               precision=hi,
                               preferred_element_type=jnp.float32)
    prespair = (present > 0.0) & (pres_row > 0.0)
    hm = jnp.where(same & offdiag & prespair, h, 0.0)
    dist_row = jnp.sum(hm, axis=1, keepdims=True)
    dist_combo = dot(McT, dist_row)
    denom = jnp.maximum((n_combo - 1.0) * n_combo, 1.0)
    dist_loss = jnp.where(n_combo < 2.0, 0.0, dist_combo / denom)
    distreg = jnp.sum(jnp.where(anyp, dist_loss + 0.001 * reg_combo, 0.0))

    batc = lax.broadcasted_iota(jnp.int32, (4, NSEG), 1)
    batr = lax.broadcasted_iota(jnp.int32, (4, NSEG), 0)
    MbT = (batc // 80 == batr).astype(jnp.float32)
    nb = dot(MbT, cnt)
    nbatch = jnp.sum((nb > 0.0).astype(jnp.float32))

    table_ref[...] = jnp.concatenate(
        [m64, wn, jnp.zeros((NSEG, W - 65), jnp.float32)], axis=1)
    svec_ref[...] = jnp.concatenate(
        [jnp.reshape(distreg, (1, 1)), jnp.reshape(nbatch, (1, 1)),
         jnp.zeros((1, 126), jnp.float32)], axis=1)


def _combine(partial):
    return pl.pallas_call(
        _combine_body,
        out_shape=(
            jax.ShapeDtypeStruct((NSEG, W), jnp.float32),
            jax.ShapeDtypeStruct((1, 128), jnp.float32),
        ),
    )(partial)


@functools.partial(
    pl.kernel,
    out_type=jax.ShapeDtypeStruct((NW * 16,), jnp.float32),
    mesh=_mesh,
    compiler_params=pltpu.CompilerParams(needs_layout_passes=False),
    scratch_types=[
        pltpu.VMEM((D * (CHUNK // 2) // 128, 128), jnp.float32),
        pltpu.VMEM((NSEG * W,), jnp.float32),
        pltpu.VMEM((8, 128), jnp.int32),
        pltpu.VMEM((16,), jnp.float32),
    ],
)
def _var_points(xT, seg_in, table, cpart, xtbuf, tbuf, seg2, obuf):
    cid = lax.axis_index("c")
    sid = lax.axis_index("s")
    wid = sid * NC + cid
    pltpu.sync_copy(table, tbuf)
    pltpu.sync_copy(seg_in.at[pl.ds(wid * 8, 8)], seg2)

    def inner(h, r, c, acc):
        segv = seg2[h * 4 + r, pl.ds(c * 16, 16)]
        fidx = segv * W
        sacc = jnp.zeros((16,), jnp.float32)
        for j in range(D):
            xj = xtbuf[j * 4 + r, pl.ds(c * 16, 16)]
            mj = plsc.load_gather(tbuf, [fidx])
            dv = xj - mj
            sacc = sacc + dv * dv
            fidx = fidx + 1
        wv = plsc.load_gather(tbuf, [fidx])
        s = sacc + 1e-12
        si = plsc.bitcast(s, jnp.int32)
        y = plsc.bitcast(lax.shift_right_logical(si, 1) + 0x1FBD1DF5,
                         jnp.float32)
        for _ in range(3):
            y = 0.5 * (y + s / y)
        hv = jnp.maximum(y - 0.5, 0.0)
        return acc + hv * hv * wv

    nrow = D * (CHUNK // 2) // 128
    varacc = jnp.zeros((16,), jnp.float32)
    for h in range(2):
        pltpu.sync_copy(xT.at[pl.ds((wid * 2 + h) * nrow, nrow)], xtbuf)

        def outer(r, acc, h=h):
            return lax.fori_loop(0, 8, lambda c, a: inner(h, r, c, a), acc)

        varacc = lax.fori_loop(0, 4, outer, varacc)
    obuf[...] = varacc
    pltpu.sync_copy(obuf, cpart.at[pl.ds(wid * 16, 16)])


def _finish_body(cpart_ref, svec_ref, out_ref):
    sv = svec_ref[...]
    total = (jnp.sum(cpart_ref[...]) + sv[0, 0]) / sv[0, 1]
    out_ref[...] = jnp.reshape(total, (1, 1))


def _finish(cpart, svec):
    return pl.pallas_call(
        _finish_body,
        out_shape=jax.ShapeDtypeStruct((1, 1), jnp.float32),
    )(cpart, svec)


@jax.jit
def kernel(out, semantic_labels, group_labels):
    x = out.astype(jnp.float32)
    xpad = jnp.concatenate(
        [x, jnp.ones((N, 1), jnp.float32), jnp.zeros((N, W - D - 1), jnp.float32)],
        axis=1)
    bidx = semantic_labels[:, 3].astype(jnp.int32)
    slab = semantic_labels[:, 4].astype(jnp.int32)
    clab = group_labels[:, 4].astype(jnp.int32)
    zer = jnp.zeros((NSEG, W), jnp.float32)
    xT = x.T.reshape(D, NW * 2, CHUNK // 2).transpose(1, 0, 2).reshape(
        NW * 2 * D * (CHUNK // 2) // 128, 128)

    partial, seg = _seg_sums(xpad, bidx, slab, clab, zer)
    table, svec = _combine(partial)
    cpart = _var_points(xT, seg, table.reshape(NSEG * W))
    res = _finish(cpart.reshape(4, 128), svec)
    return res[0, 0]


# speedup vs baseline: 8.3248x; 1.0075x over previous
import functools
import jax
import jax.numpy as jnp
from jax import lax
from jax.experimental import pallas as pl
from jax.experimental.pallas import tpu as pltpu
from jax.experimental.pallas import tpu_sc as plsc

N = 32768
D = 64
W = 80
NSEG = 320
NC = 2
NS = 16
NW = NC * NS
CHUNK = N // NW
GROUPS = CHUNK // 16

_mesh = plsc.VectorSubcoreMesh(
    core_axis_name="c", subcore_axis_name="s", num_cores=NC, num_subcores=NS)


@functools.partial(
    pl.kernel,
    out_type=(
        jax.ShapeDtypeStruct((NC * NSEG, W), jnp.float32),
        jax.ShapeDtypeStruct((N // 128, 128), jnp.int32),
    ),
    mesh=_mesh,
    scratch_types=[
        pltpu.VMEM((CHUNK // 2, W), jnp.float32),
        pltpu.VMEM((CHUNK,), jnp.int32),
        pltpu.VMEM((CHUNK,), jnp.int32),
        pltpu.VMEM((CHUNK,), jnp.int32),
        pltpu.VMEM((8, 128), jnp.int32),
        pltpu.VMEM_SHARED((NSEG, W), jnp.float32),
        pltpu.SemaphoreType.DMA,
    ],
)
def _seg_sums(xpad, bidx, slab, clab, zer, partial, seg_out,
              xbuf, bbuf, sbuf, cbuf, seg2, shared, sem):
    cid = lax.axis_index("c")
    sid = lax.axis_index("s")
    wid = sid * NC + cid
    base = wid * CHUNK
    pltpu.sync_copy(bidx.at[pl.ds(base, CHUNK)], bbuf)
    pltpu.sync_copy(slab.at[pl.ds(base, CHUNK)], sbuf)
    pltpu.sync_copy(clab.at[pl.ds(base, CHUNK)], cbuf)
    @pl.when(sid < 10)
    def _zero():
        pltpu.sync_copy(zer.at[pl.ds(sid * 32, 32)],
                        shared.at[pl.ds(sid * 32, 32)])
    for i in range(GROUPS):
        b = bbuf[pl.ds(i * 16, 16)]
        s = sbuf[pl.ds(i * 16, 16)]
        g = cbuf[pl.ds(i * 16, 16)]
        seg2[i // 8, pl.ds((i % 8) * 16, 16)] = b * 80 + s * 20 + g
    pltpu.sync_copy(seg2, seg_out.at[pl.ds(wid * 8, 8)])
    plsc.subcore_barrier()
    for h in range(2):
        pltpu.sync_copy(xpad.at[pl.ds(base + h * (CHUNK // 2), CHUNK // 2)],
                        xbuf)
        descs = [pltpu.async_copy(xbuf.at[pl.ds(k * 128, 128)],
                                  shared.at[seg2.at[h * 4 + k]], sem,
                                  add=True)
                 for k in range(4)]
        for dsc in descs:
            dsc.wait()
    plsc.subcore_barrier()

    @pl.when(sid < 10)
    def _writeout():
        pltpu.sync_copy(shared.at[pl.ds(sid * 32, 32)],
                        partial.at[pl.ds(cid * NSEG + sid * 32, 32)])


def _combine_body(partial_ref, table_ref, svec_ref):
    p = partial_ref[...]
    sums = p[0:NSEG] + p[NSEG:2 * NSEG]
    cnt = sums[:, 64:65]
    safe_cnt = jnp.maximum(cnt, 1.0)
    means80 = sums / safe_cnt
    m64 = means80[:, 0:64]
    present = (cnt > 0.0).astype(jnp.float32)

    segr = lax.broadcasted_iota(jnp.int32, (NSEG, 16), 0)
    comc = lax.broadcasted_iota(jnp.int32, (NSEG, 16), 1)
    Mc = (segr // 20 == comc).astype(jnp.float32)
    segc = lax.broadcasted_iota(jnp.int32, (16, NSEG), 1)
    comr = lax.broadcasted_iota(jnp.int32, (16, NSEG), 0)
    McT = (segc // 20 == comr).astype(jnp.float32)

    dn = (((1,), (0,)), ((), ()))
    hi = jax.lax.Precision.HIGHEST
    dot = functools.partial(lax.dot_general, dimension_numbers=dn,
                            precision=hi, preferred_element_type=jnp.float32)
    n_combo = dot(McT, present)
    safe_n = jnp.maximum(n_combo, 1.0)
    cnt_combo = dot(McT, cnt)
    anyp = cnt_combo > 0.0
    n_seg = dot(Mc, n_combo)
    wn = jnp.where(present > 0.0,
                   1.0 / (safe_cnt * jnp.maximum(n_seg, 1.0)), 0.0)

    m2 = m64 * m64
    sqn = jnp.sum(m2, axis=1, keepdims=True)
    sqn_row = lax.dot_general(jnp.ones((1, 64), jnp.float32), m2,
                              dimension_numbers=(((1,), (1,)), ((), ())),
                              precision=hi,
                              preferred_element_type=jnp.float32)
    regseg = jnp.sqrt(sqn + 1e-12) * present
    reg_combo = dot(McT, regseg) / safe_n

    G = lax.dot_general(m64, m64,
                        dimension_numbers=(((1,), (1,)), ((), ())),
                        precision=hi,
                        preferred_element_type=jnp.float32)
    pair_sq = jnp.maximum(sqn + sqn_row - 2.0 * G, 0.0)
    d = jnp.sqrt(pair_sq + 1e-12)
    h = jnp.maximum(3.0 - d, 0.0) ** 2
    ri = lax.broadcasted_iota(jnp.int32, (NSEG, NSEG), 0)
    ci = lax.broadcasted_iota(jnp.int32, (NSEG, NSEG), 1)
    same = (ri // 20) == (ci // 20)
    offdiag = ri != ci
    pres_row = lax.dot_general(jnp.ones((1, 1), jnp.float32), present,
                               dimension_numbers=(((1,), (1,)), ((), ())),
                               precision=hi,
                               preferred_element_type=jnp.float32)
    prespair = (present > 0.0) & (pres_row > 0.0)
    hm = jnp.where(same & offdiag & prespair, h, 0.0)
    dist_row = jnp.sum(hm, axis=1, keepdims=True)
    dist_combo = dot(McT, dist_row)
    denom = jnp.maximum((n_combo - 1.0) * n_combo, 1.0)
    dist_loss = jnp.where(n_combo < 2.0, 0.0, dist_combo / denom)
    distreg = jnp.sum(jnp.where(anyp, dist_loss + 0.001 * reg_combo, 0.0))

    batc = lax.broadcasted_iota(jnp.int32, (4, NSEG), 1)
    batr = lax.broadcasted_iota(jnp.int32, (4, NSEG), 0)
    MbT = (batc // 80 == batr).astype(jnp.float32)
    nb = dot(MbT, cnt)
    nbatch = jnp.sum((nb > 0.0).astype(jnp.float32))

    table_ref[...] = jnp.concatenate(
        [m64, wn, jnp.zeros((NSEG, W - 65), jnp.float32)], axis=1)
    svec_ref[...] = jnp.concatenate(
        [jnp.reshape(distreg, (1, 1)), jnp.reshape(nbatch, (1, 1)),
         jnp.zeros((1, 126), jnp.float32)], axis=1)


def _combine(partial):
    return pl.pallas_call(
        _combine_body,
        out_shape=(
            jax.ShapeDtypeStruct((NSEG, W), jnp.float32),
            jax.ShapeDtypeStruct((1, 128), jnp.float32),
        ),
    )(partial)


@functools.partial(
    pl.kernel,
    out_type=jax.ShapeDtypeStruct((NW * 16,), jnp.float32),
    mesh=_mesh,
    compiler_params=pltpu.CompilerParams(needs_layout_passes=False),
    scratch_types=[
        pltpu.VMEM((D * (CHUNK // 2) // 128, 128), jnp.float32),
        pltpu.VMEM((NSEG * W,), jnp.float32),
        pltpu.VMEM((8, 128), jnp.int32),
        pltpu.VMEM((16,), jnp.float32),
        pltpu.SemaphoreType.DMA,
    ],
)
def _var_points(xT, seg_in, table, cpart, xtbuf, tbuf, seg2, obuf, sem):
    cid = lax.axis_index("c")
    sid = lax.axis_index("s")
    wid = sid * NC + cid
    d1 = pltpu.async_copy(table, tbuf, sem)
    d2 = pltpu.async_copy(seg_in.at[pl.ds(wid * 8, 8)], seg2, sem)

    def inner(h, r, c, acc):
        segv = seg2[h * 4 + r, pl.ds(c * 16, 16)]
        fidx = segv * W
        sacc = jnp.zeros((16,), jnp.float32)
        for j in range(D):
            xj = xtbuf[j * 4 + r, pl.ds(c * 16, 16)]
            mj = plsc.load_gather(tbuf, [fidx])
            dv = xj - mj
            sacc = sacc + dv * dv
            fidx = fidx + 1
        wv = plsc.load_gather(tbuf, [fidx])
        s = sacc + 1e-12
        si = plsc.bitcast(s, jnp.int32)
        y = plsc.bitcast(lax.shift_right_logical(si, 1) + 0x1FBD1DF5,
                         jnp.float32)
        for _ in range(3):
            y = 0.5 * (y + s / y)
        hv = jnp.maximum(y - 0.5, 0.0)
        return acc + hv * hv * wv

    nrow = D * (CHUNK // 2) // 128
    varacc = jnp.zeros((16,), jnp.float32)
    for h in range(2):
        pltpu.sync_copy(xT.at[pl.ds((wid * 2 + h) * nrow, nrow)], xtbuf)
        if h == 0:
            d1.wait()
            d2.wait()

        def outer(r, acc, h=h):
            return lax.fori_loop(0, 8, lambda c, a: inner(h, r, c, a), acc)

        varacc = lax.fori_loop(0, 4, outer, varacc)
    obuf[...] = varacc
    pltpu.sync_copy(obuf, cpart.at[pl.ds(wid * 16, 16)])


def _finish_body(cpart_ref, svec_ref, out_ref):
    sv = svec_ref[...]
    total = (jnp.sum(cpart_ref[...]) + sv[0, 0]) / sv[0, 1]
    out_ref[...] = jnp.reshape(total, (1, 1))


def _finish(cpart, svec):
    return pl.pallas_call(
        _finish_body,
        out_shape=jax.ShapeDtypeStruct((1, 1), jnp.float32),
    )(cpart, svec)


@jax.jit
def kernel(out, semantic_labels, group_labels):
    x = out.astype(jnp.float32)
    xpad = jnp.concatenate(
        [x, jnp.ones((N, 1), jnp.float32), jnp.zeros((N, W - D - 1), jnp.float32)],
        axis=1)
    bidx = semantic_labels[:, 3].astype(jnp.int32)
    slab = semantic_labels[:, 4].astype(jnp.int32)
    clab = group_labels[:, 4].astype(jnp.int32)
    zer = jnp.zeros((NSEG, W), jnp.float32)
    xT = x.T.reshape(D, NW * 2, CHUNK // 2).transpose(1, 0, 2).reshape(
        NW * 2 * D * (CHUNK // 2) // 128, 128)

    partial, seg = _seg_sums(xpad, bidx, slab, clab, zer)
    table, svec = _combine(partial)
    cpart = _var_points(xT, seg, table.reshape(NSEG * W))
    res = _finish(cpart.reshape(4, 128), svec)
    return res[0, 0]


# speedup vs baseline: 8.4647x; 1.0168x over previous
import functools
import jax
import jax.numpy as jnp
from jax import lax
from jax.experimental import pallas as pl
from jax.experimental.pallas import tpu as pltpu
from jax.experimental.pallas import tpu_sc as plsc

N = 32768
D = 64
W = 80
NSEG = 320
NC = 2
NS = 16
NW = NC * NS
CHUNK = N // NW
GROUPS = CHUNK // 16

_mesh = plsc.VectorSubcoreMesh(
    core_axis_name="c", subcore_axis_name="s", num_cores=NC, num_subcores=NS)


@functools.partial(
    pl.kernel,
    out_type=(
        jax.ShapeDtypeStruct((NC * NSEG, W), jnp.float32),
        jax.ShapeDtypeStruct((N // 128, 128), jnp.int32),
    ),
    mesh=_mesh,
    scratch_types=[
        pltpu.VMEM((CHUNK // 2, W), jnp.float32),
        pltpu.VMEM((CHUNK,), jnp.int32),
        pltpu.VMEM((CHUNK,), jnp.int32),
        pltpu.VMEM((CHUNK,), jnp.int32),
        pltpu.VMEM((8, 128), jnp.int32),
        pltpu.VMEM_SHARED((NSEG, W), jnp.float32),
        pltpu.SemaphoreType.DMA,
    ],
)
def _seg_sums(xpad, bidx, slab, clab, zer, partial, seg_out,
              xbuf, bbuf, sbuf, cbuf, seg2, shared, sem):
    cid = lax.axis_index("c")
    sid = lax.axis_index("s")
    wid = sid * NC + cid
    base = wid * CHUNK
    d1 = pltpu.async_copy(bidx.at[pl.ds(base, CHUNK)], bbuf, sem)
    d2 = pltpu.async_copy(slab.at[pl.ds(base, CHUNK)], sbuf, sem)
    d3 = pltpu.async_copy(clab.at[pl.ds(base, CHUNK)], cbuf, sem)
    @pl.when(sid < 10)
    def _zero():
        pltpu.sync_copy(zer.at[pl.ds(sid * 32, 32)],
                        shared.at[pl.ds(sid * 32, 32)])
    d1.wait()
    d2.wait()
    d3.wait()
    for i in range(GROUPS):
        b = bbuf[pl.ds(i * 16, 16)]
        s = sbuf[pl.ds(i * 16, 16)]
        g = cbuf[pl.ds(i * 16, 16)]
        seg2[i // 8, pl.ds((i % 8) * 16, 16)] = b * 80 + s * 20 + g
    pltpu.sync_copy(seg2, seg_out.at[pl.ds(wid * 8, 8)])
    plsc.subcore_barrier()
    for h in range(2):
        pltpu.sync_copy(xpad.at[pl.ds(base + h * (CHUNK // 2), CHUNK // 2)],
                        xbuf)
        descs = [pltpu.async_copy(xbuf.at[pl.ds(k * 128, 128)],
                                  shared.at[seg2.at[h * 4 + k]], sem,
                                  add=True)
                 for k in range(4)]
        for dsc in descs:
            dsc.wait()
    plsc.subcore_barrier()

    @pl.when(sid < 10)
    def _writeout():
        pltpu.sync_copy(shared.at[pl.ds(sid * 32, 32)],
                        partial.at[pl.ds(cid * NSEG + sid * 32, 32)])


def _combine_body(partial_ref, table_ref, svec_ref):
    p = partial_ref[...]
    sums = p[0:NSEG] + p[NSEG:2 * NSEG]
    cnt = sums[:, 64:65]
    safe_cnt = jnp.maximum(cnt, 1.0)
    means80 = sums / safe_cnt
    m64 = means80[:, 0:64]
    present = (cnt > 0.0).astype(jnp.float32)

    segr = lax.broadcasted_iota(jnp.int32, (NSEG, 16), 0)
    comc = lax.broadcasted_iota(jnp.int32, (NSEG, 16), 1)
    Mc = (segr // 20 == comc).astype(jnp.float32)
    segc = lax.broadcasted_iota(jnp.int32, (16, NSEG), 1)
    comr = lax.broadcasted_iota(jnp.int32, (16, NSEG), 0)
    McT = (segc // 20 == comr).astype(jnp.float32)

    dn = (((1,), (0,)), ((), ()))
    hi = jax.lax.Precision.HIGHEST
    dot = functools.partial(lax.dot_general, dimension_numbers=dn,
                            precision=hi, preferred_element_type=jnp.float32)
    n_combo = dot(McT, present)
    safe_n = jnp.maximum(n_combo, 1.0)
    cnt_combo = dot(McT, cnt)
    anyp = cnt_combo > 0.0
    n_seg = dot(Mc, n_combo)
    wn = jnp.where(present > 0.0,
                   1.0 / (safe_cnt * jnp.maximum(n_seg, 1.0)), 0.0)

    m2 = m64 * m64
    sqn = jnp.sum(m2, axis=1, keepdims=True)
    sqn_row = lax.dot_general(jnp.ones((1, 64), jnp.float32), m2,
                              dimension_numbers=(((1,), (1,)), ((), ())),
                              precision=hi,
                              preferred_element_type=jnp.float32)
    regseg = jnp.sqrt(sqn + 1e-12) * present
    reg_combo = dot(McT, regseg) / safe_n

    G = lax.dot_general(m64, m64,
                        dimension_numbers=(((1,), (1,)), ((), ())),
                        precision=hi,
                        preferred_element_type=jnp.float32)
    pair_sq = jnp.maximum(sqn + sqn_row - 2.0 * G, 0.0)
    d = jnp.sqrt(pair_sq + 1e-12)
    h = jnp.maximum(3.0 - d, 0.0) ** 2
    ri = lax.broadcasted_iota(jnp.int32, (NSEG, NSEG), 0)
    ci = lax.broadcasted_iota(jnp.int32, (NSEG, NSEG), 1)
    same = (ri // 20) == (ci // 20)
    offdiag = ri != ci
    pres_row = lax.dot_general(jnp.ones((1, 1), jnp.float32), present,
                               dimension_numbers=(((1,), (1,)), ((), ())),
                               precision=hi,
                               preferred_element_type=jnp.float32)
    prespair = (present > 0.0) & (pres_row > 0.0)
    hm = jnp.where(same & offdiag & prespair, h, 0.0)
    dist_row = jnp.sum(hm, axis=1, keepdims=True)
    dist_combo = dot(McT, dist_row)
    denom = jnp.maximum((n_combo - 1.0) * n_combo, 1.0)
    dist_loss = jnp.where(n_combo < 2.0, 0.0, dist_combo / denom)
    distreg = jnp.sum(jnp.where(anyp, dist_loss + 0.001 * reg_combo, 0.0))

    batc = lax.broadcasted_iota(jnp.int32, (4, NSEG), 1)
    batr = lax.broadcasted_iota(jnp.int32, (4, NSEG), 0)
    MbT = (batc // 80 == batr).astype(jnp.float32)
    nb = dot(MbT, cnt)
    nbatch = jnp.sum((nb > 0.0).astype(jnp.float32))

    table_ref[...] = jnp.concatenate(
        [m64, wn, jnp.zeros((NSEG, W - 65), jnp.float32)], axis=1)
    svec_ref[...] = jnp.concatenate(
        [jnp.reshape(distreg, (1, 1)), jnp.reshape(nbatch, (1, 1)),
         jnp.zeros((1, 126), jnp.float32)], axis=1)


def _combine(partial):
    return pl.pallas_call(
        _combine_body,
        out_shape=(
            jax.ShapeDtypeStruct((NSEG, W), jnp.float32),
            jax.ShapeDtypeStruct((1, 128), jnp.float32),
        ),
    )(partial)


@functools.partial(
    pl.kernel,
    out_type=jax.ShapeDtypeStruct((NW * 16,), jnp.float32),
    mesh=_mesh,
    compiler_params=pltpu.CompilerParams(needs_layout_passes=False),
    scratch_types=[
        pltpu.VMEM((D * (CHUNK // 2) // 128, 128), jnp.float32),
        pltpu.VMEM((NSEG * W,), jnp.float32),
        pltpu.VMEM((8, 128), jnp.int32),
        pltpu.VMEM((16,), jnp.float32),
        pltpu.SemaphoreType.DMA,
    ],
)
def _var_points(xT, seg_in, table, cpart, xtbuf, tbuf, seg2, obuf, sem):
    cid = lax.axis_index("c")
    sid = lax.axis_index("s")
    wid = sid * NC + cid
    d1 = pltpu.async_copy(table, tbuf, sem)
    d2 = pltpu.async_copy(seg_in.at[pl.ds(wid * 8, 8)], seg2, sem)

    def inner(h, r, c, acc):
        segv = seg2[h * 4 + r, pl.ds(c * 16, 16)]
        fidx = segv * W
        sacc = jnp.zeros((16,), jnp.float32)
        for j in range(D):
            xj = xtbuf[j * 4 + r, pl.ds(c * 16, 16)]
            mj = plsc.load_gather(tbuf, [fidx])
            dv = xj - mj
            sacc = sacc + dv * dv
            fidx = fidx + 1
        wv = plsc.load_gather(tbuf, [fidx])
        s = sacc + 1e-12
        si = plsc.bitcast(s, jnp.int32)
        y = plsc.bitcast(lax.shift_right_logical(si, 1) + 0x1FBD1DF5,
                         jnp.float32)
        for _ in range(3):
            y = 0.5 * (y + s / y)
        hv = jnp.maximum(y - 0.5, 0.0)
        return acc + hv * hv * wv

    nrow = D * (CHUNK // 2) // 128
    varacc = jnp.zeros((16,), jnp.float32)
    for h in range(2):
        pltpu.sync_copy(xT.at[pl.ds((wid * 2 + h) * nrow, nrow)], xtbuf)
        if h == 0:
            d1.wait()
            d2.wait()

        def outer(r, acc, h=h):
            return lax.fori_loop(0, 8, lambda c, a: inner(h, r, c, a), acc)

        varacc = lax.fori_loop(0, 4, outer, varacc)
    obuf[...] = varacc
    pltpu.sync_copy(obuf, cpart.at[pl.ds(wid * 16, 16)])


def _finish_body(cpart_ref, svec_ref, out_ref):
    sv = svec_ref[...]
    total = (jnp.sum(cpart_ref[...]) + sv[0, 0]) / sv[0, 1]
    out_ref[...] = jnp.reshape(total, (1, 1))


def _finish(cpart, svec):
    return pl.pallas_call(
        _finish_body,
        out_shape=jax.ShapeDtypeStruct((1, 1), jnp.float32),
    )(cpart, svec)


@jax.jit
def kernel(out, semantic_labels, group_labels):
    x = out.astype(jnp.float32)
    xpad = jnp.concatenate(
        [x, jnp.ones((N, 1), jnp.float32), jnp.zeros((N, W - D - 1), jnp.float32)],
        axis=1)
    bidx = semantic_labels[:, 3].astype(jnp.int32)
    slab = semantic_labels[:, 4].astype(jnp.int32)
    clab = group_labels[:, 4].astype(jnp.int32)
    zer = jnp.zeros((NSEG, W), jnp.float32)
    xT = x.T.reshape(D, NW * 2, CHUNK // 2).transpose(1, 0, 2).reshape(
        NW * 2 * D * (CHUNK // 2) // 128, 128)

    partial, seg = _seg_sums(xpad, bidx, slab, clab, zer)
    table, svec = _combine(partial)
    cpart = _var_points(xT, seg, table.reshape(NSEG * W))
    res = _finish(cpart.reshape(4, 128), svec)
    return res[0, 0]


# speedup vs baseline: 8.6129x; 1.0175x over previous
import functools
import jax
import jax.numpy as jnp
from jax import lax
from jax.experimental import pallas as pl
from jax.experimental.pallas import tpu as pltpu
from jax.experimental.pallas import tpu_sc as plsc

N = 32768
D = 64
W = 80
NSEG = 320
NC = 2
NS = 16
NW = NC * NS
CHUNK = N // NW
GROUPS = CHUNK // 16

_mesh = plsc.VectorSubcoreMesh(
    core_axis_name="c", subcore_axis_name="s", num_cores=NC, num_subcores=NS)


@functools.partial(
    pl.kernel,
    out_type=(
        jax.ShapeDtypeStruct((NC * NSEG, W), jnp.float32),
        jax.ShapeDtypeStruct((N // 128, 128), jnp.int32),
    ),
    mesh=_mesh,
    scratch_types=[
        pltpu.VMEM((CHUNK // 2, W), jnp.float32),
        pltpu.VMEM((CHUNK,), jnp.int32),
        pltpu.VMEM((CHUNK,), jnp.int32),
        pltpu.VMEM((CHUNK,), jnp.int32),
        pltpu.VMEM((8, 128), jnp.int32),
        pltpu.VMEM_SHARED((NSEG, W), jnp.float32),
        pltpu.SemaphoreType.DMA,
    ],
)
def _seg_sums(xpad, bidx, slab, clab, zer, partial, seg_out,
              xbuf, bbuf, sbuf, cbuf, seg2, shared, sem):
    cid = lax.axis_index("c")
    sid = lax.axis_index("s")
    wid = sid * NC + cid
    base = wid * CHUNK
    d1 = pltpu.async_copy(bidx.at[pl.ds(base, CHUNK)], bbuf, sem)
    d2 = pltpu.async_copy(slab.at[pl.ds(base, CHUNK)], sbuf, sem)
    d3 = pltpu.async_copy(clab.at[pl.ds(base, CHUNK)], cbuf, sem)
    @pl.when(sid < 10)
    def _zero():
        pltpu.sync_copy(zer.at[pl.ds(sid * 32, 32)],
                        shared.at[pl.ds(sid * 32, 32)])
    d1.wait()
    d2.wait()
    d3.wait()
    for i in range(GROUPS):
        b = bbuf[pl.ds(i * 16, 16)]
        s = sbuf[pl.ds(i * 16, 16)]
        g = cbuf[pl.ds(i * 16, 16)]
        seg2[i // 8, pl.ds((i % 8) * 16, 16)] = b * 80 + s * 20 + g
    pltpu.sync_copy(seg2, seg_out.at[pl.ds(wid * 8, 8)])
    plsc.subcore_barrier()
    for h in range(2):
        pltpu.sync_copy(xpad.at[pl.ds(base + h * (CHUNK // 2), CHUNK // 2)],
                        xbuf)
        descs = [pltpu.async_copy(xbuf.at[pl.ds(k * 128, 128)],
                                  shared.at[seg2.at[h * 4 + k]], sem,
                                  add=True)
                 for k in range(4)]
        for dsc in descs:
            dsc.wait()
    plsc.subcore_barrier()

    @pl.when(sid < 10)
    def _writeout():
        pltpu.sync_copy(shared.at[pl.ds(sid * 32, 32)],
                        partial.at[pl.ds(cid * NSEG + sid * 32, 32)])


def _combine_body(partial_ref, table_ref, svec_ref):
    p = partial_ref[...]
    sums = p[0:NSEG] + p[NSEG:2 * NSEG]
    cnt = sums[:, 64:65]
    safe_cnt = jnp.maximum(cnt, 1.0)
    means80 = sums / safe_cnt
    m64 = means80[:, 0:64]
    present = (cnt > 0.0).astype(jnp.float32)

    segr = lax.broadcasted_iota(jnp.int32, (NSEG, 16), 0)
    comc = lax.broadcasted_iota(jnp.int32, (NSEG, 16), 1)
    Mc = (segr // 20 == comc).astype(jnp.float32)
    segc = lax.broadcasted_iota(jnp.int32, (16, NSEG), 1)
    comr = lax.broadcasted_iota(jnp.int32, (16, NSEG), 0)
    McT = (segc // 20 == comr).astype(jnp.float32)

    dn = (((1,), (0,)), ((), ()))
    hi = jax.lax.Precision.HIGHEST
    dot = functools.partial(lax.dot_general, dimension_numbers=dn,
                            precision=hi, preferred_element_type=jnp.float32)
    n_combo = dot(McT, present)
    safe_n = jnp.maximum(n_combo, 1.0)
    cnt_combo = dot(McT, cnt)
    anyp = cnt_combo > 0.0
    n_seg = dot(Mc, n_combo)
    wn = jnp.where(present > 0.0,
                   1.0 / (safe_cnt * jnp.maximum(n_seg, 1.0)), 0.0)

    m2 = m64 * m64
    sqn = jnp.sum(m2, axis=1, keepdims=True)
    sqn_row = lax.dot_general(jnp.ones((1, 64), jnp.float32), m2,
                              dimension_numbers=(((1,), (1,)), ((), ())),
                              precision=hi,
                              preferred_element_type=jnp.float32)
    regseg = jnp.sqrt(sqn + 1e-12) * present
    reg_combo = dot(McT, regseg) / safe_n

    G = lax.dot_general(m64, m64,
                        dimension_numbers=(((1,), (1,)), ((), ())),
                        precision=hi,
                        preferred_element_type=jnp.float32)
    pair_sq = jnp.maximum(sqn + sqn_row - 2.0 * G, 0.0)
    d = jnp.sqrt(pair_sq + 1e-12)
    h = jnp.maximum(3.0 - d, 0.0) ** 2
    ri = lax.broadcasted_iota(jnp.int32, (NSEG, NSEG), 0)
    ci = lax.broadcasted_iota(jnp.int32, (NSEG, NSEG), 1)
    same = (ri // 20) == (ci // 20)
    offdiag = ri != ci
    pres_row = lax.dot_general(jnp.ones((1, 1), jnp.float32), present,
                               dimension_numbers=(((1,), (1,)), ((), ())),
                               precision=hi,
                               preferred_element_type=jnp.float32)
    prespair = (present > 0.0) & (pres_row > 0.0)
    hm = jnp.where(same & offdiag & prespair, h, 0.0)
    dist_row = jnp.sum(hm, axis=1, keepdims=True)
    dist_combo = dot(McT, dist_row)
    denom = jnp.maximum((n_combo - 1.0) * n_combo, 1.0)
    dist_loss = jnp.where(n_combo < 2.0, 0.0, dist_combo / denom)
    distreg = jnp.sum(jnp.where(anyp, dist_loss + 0.001 * reg_combo, 0.0))

    batc = lax.broadcasted_iota(jnp.int32, (4, NSEG), 1)
    batr = lax.broadcasted_iota(jnp.int32, (4, NSEG), 0)
    MbT = (batc // 80 == batr).astype(jnp.float32)
    nb = dot(MbT, cnt)
    nbatch = jnp.sum((nb > 0.0).astype(jnp.float32))

    table_ref[...] = jnp.concatenate(
        [m64, wn, jnp.zeros((NSEG, W - 65), jnp.float32)], axis=1)
    svec_ref[...] = jnp.concatenate(
        [jnp.reshape(distreg, (1, 1)), jnp.reshape(nbatch, (1, 1)),
         jnp.zeros((1, 126), jnp.float32)], axis=1)


def _combine(partial):
    return pl.pallas_call(
        _combine_body,
        out_shape=(
            jax.ShapeDtypeStruct((NSEG, W), jnp.float32),
            jax.ShapeDtypeStruct((1, 128), jnp.float32),
        ),
    )(partial)


@functools.partial(
    pl.kernel,
    out_type=jax.ShapeDtypeStruct((NW * 16,), jnp.float32),
    mesh=_mesh,
    compiler_params=pltpu.CompilerParams(needs_layout_passes=False),
    scratch_types=[
        pltpu.VMEM((2, D * (CHUNK // 2) // 128, 128), jnp.float32),
        pltpu.VMEM((NSEG * W,), jnp.float32),
        pltpu.VMEM((8, 128), jnp.int32),
        pltpu.VMEM((16,), jnp.float32),
        pltpu.SemaphoreType.DMA,
    ],
)
def _var_points(xT, seg_in, table, cpart, xtbuf, tbuf, seg2, obuf, sem):
    cid = lax.axis_index("c")
    sid = lax.axis_index("s")
    wid = sid * NC + cid
    d1 = pltpu.async_copy(table, tbuf, sem)
    d2 = pltpu.async_copy(seg_in.at[pl.ds(wid * 8, 8)], seg2, sem)

    def inner(h, r, c, acc):
        segv = seg2[h * 4 + r, pl.ds(c * 16, 16)]
        fidx = segv * W
        sacc = jnp.zeros((16,), jnp.float32)
        for j in range(D):
            xj = xtbuf[h, j * 4 + r, pl.ds(c * 16, 16)]
            mj = plsc.load_gather(tbuf, [fidx])
            dv = xj - mj
            sacc = sacc + dv * dv
            fidx = fidx + 1
        wv = plsc.load_gather(tbuf, [fidx])
        s = sacc + 1e-12
        si = plsc.bitcast(s, jnp.int32)
        y = plsc.bitcast(lax.shift_right_logical(si, 1) + 0x1FBD1DF5,
                         jnp.float32)
        for _ in range(3):
            y = 0.5 * (y + s / y)
        hv = jnp.maximum(y - 0.5, 0.0)
        return acc + hv * hv * wv

    nrow = D * (CHUNK // 2) // 128
    varacc = jnp.zeros((16,), jnp.float32)
    rd0 = pltpu.async_copy(xT.at[pl.ds(wid * 2 * nrow, nrow)],
                           xtbuf.at[0], sem)
    rd0.wait()
    d1.wait()
    d2.wait()
    rd1 = pltpu.async_copy(xT.at[pl.ds((wid * 2 + 1) * nrow, nrow)],
                           xtbuf.at[1], sem)
    for h in range(2):
        if h == 1:
            rd1.wait()

        def outer(r, acc, h=h):
            return lax.fori_loop(0, 8, lambda c, a: inner(h, r, c, a), acc)

        varacc = lax.fori_loop(0, 4, outer, varacc)
    obuf[...] = varacc
    pltpu.sync_copy(obuf, cpart.at[pl.ds(wid * 16, 16)])


def _finish_body(cpart_ref, svec_ref, out_ref):
    sv = svec_ref[...]
    total = (jnp.sum(cpart_ref[...]) + sv[0, 0]) / sv[0, 1]
    out_ref[...] = jnp.reshape(total, (1, 1))


def _finish(cpart, svec):
    return pl.pallas_call(
        _finish_body,
        out_shape=jax.ShapeDtypeStruct((1, 1), jnp.float32),
    )(cpart, svec)


@jax.jit
def kernel(out, semantic_labels, group_labels):
    x = out.astype(jnp.float32)
    xpad = jnp.concatenate(
        [x, jnp.ones((N, 1), jnp.float32), jnp.zeros((N, W - D - 1), jnp.float32)],
        axis=1)
    bidx = semantic_labels[:, 3].astype(jnp.int32)
    slab = semantic_labels[:, 4].astype(jnp.int32)
    clab = group_labels[:, 4].astype(jnp.int32)
    zer = jnp.zeros((NSEG, W), jnp.float32)
    xT = x.T.reshape(D, NW * 2, CHUNK // 2).transpose(1, 0, 2).reshape(
        NW * 2 * D * (CHUNK // 2) // 128, 128)

    partial, seg = _seg_sums(xpad, bidx, slab, clab, zer)
    table, svec = _combine(partial)
    cpart = _var_points(xT, seg, table.reshape(NSEG * W))
    res = _finish(cpart.reshape(4, 128), svec)
    return res[0, 0]


# speedup vs baseline: 8.7855x; 1.0200x over previous
import functools
import jax
import jax.numpy as jnp
from jax import lax
from jax.experimental import pallas as pl
from jax.experimental.pallas import tpu as pltpu
from jax.experimental.pallas import tpu_sc as plsc

N = 32768
D = 64
W = 80
NSEG = 320
NC = 2
NS = 16
NW = NC * NS
CHUNK = N // NW
GROUPS = CHUNK // 16

_mesh = plsc.VectorSubcoreMesh(
    core_axis_name="c", subcore_axis_name="s", num_cores=NC, num_subcores=NS)


@functools.partial(
    pl.kernel,
    out_type=(
        jax.ShapeDtypeStruct((NC * NSEG, W), jnp.float32),
        jax.ShapeDtypeStruct((N // 128, 128), jnp.int32),
    ),
    mesh=_mesh,
    scratch_types=[
        pltpu.VMEM((2, CHUNK // 4, W), jnp.float32),
        pltpu.VMEM((CHUNK,), jnp.int32),
        pltpu.VMEM((CHUNK,), jnp.int32),
        pltpu.VMEM((CHUNK,), jnp.int32),
        pltpu.VMEM((8, 128), jnp.int32),
        pltpu.VMEM_SHARED((NSEG, W), jnp.float32),
        pltpu.SemaphoreType.DMA,
        pltpu.SemaphoreType.DMA,
        pltpu.SemaphoreType.DMA,
    ],
)
def _seg_sums(xpad, bidx, slab, clab, zer, partial, seg_out,
              xbuf, bbuf, sbuf, cbuf, seg2, shared, sem, sem_rd, sem_s1):
    cid = lax.axis_index("c")
    sid = lax.axis_index("s")
    wid = sid * NC + cid
    base = wid * CHUNK
    d1 = pltpu.async_copy(bidx.at[pl.ds(base, CHUNK)], bbuf, sem)
    d2 = pltpu.async_copy(slab.at[pl.ds(base, CHUNK)], sbuf, sem)
    d3 = pltpu.async_copy(clab.at[pl.ds(base, CHUNK)], cbuf, sem)
    @pl.when(sid < 10)
    def _zero():
        pltpu.sync_copy(zer.at[pl.ds(sid * 32, 32)],
                        shared.at[pl.ds(sid * 32, 32)])
    d1.wait()
    d2.wait()
    d3.wait()
    for i in range(GROUPS):
        b = bbuf[pl.ds(i * 16, 16)]
        s = sbuf[pl.ds(i * 16, 16)]
        g = cbuf[pl.ds(i * 16, 16)]
        seg2[i // 8, pl.ds((i % 8) * 16, 16)] = b * 80 + s * 20 + g
    pltpu.sync_copy(seg2, seg_out.at[pl.ds(wid * 8, 8)])
    plsc.subcore_barrier()
    QP = CHUNK // 4
    rd = {0: pltpu.async_copy(xpad.at[pl.ds(base, QP)], xbuf.at[0], sem_rd)}
    scat = {}
    for q in range(4):
        buf = q % 2
        rd[q].wait()
        ssem = sem if buf == 0 else sem_s1
        scat[q] = [pltpu.async_copy(xbuf.at[buf, pl.ds(k * 128, 128)],
                                    shared.at[seg2.at[q * 2 + k]], ssem,
                                    add=True)
                   for k in range(2)]
        if q < 3:
            if q >= 1:
                for dsc in scat[q - 1]:
                    dsc.wait()
            rd[q + 1] = pltpu.async_copy(
                xpad.at[pl.ds(base + (q + 1) * QP, QP)],
                xbuf.at[1 - buf], sem_rd)
    for q in (2, 3):
        for dsc in scat[q]:
            dsc.wait()
    plsc.subcore_barrier()

    @pl.when(sid < 10)
    def _writeout():
        pltpu.sync_copy(shared.at[pl.ds(sid * 32, 32)],
                        partial.at[pl.ds(cid * NSEG + sid * 32, 32)])


def _combine_body(partial_ref, table_ref, svec_ref):
    p = partial_ref[...]
    sums = p[0:NSEG] + p[NSEG:2 * NSEG]
    cnt = sums[:, 64:65]
    safe_cnt = jnp.maximum(cnt, 1.0)
    means80 = sums / safe_cnt
    m64 = means80[:, 0:64]
    present = (cnt > 0.0).astype(jnp.float32)

    segr = lax.broadcasted_iota(jnp.int32, (NSEG, 16), 0)
    comc = lax.broadcasted_iota(jnp.int32, (NSEG, 16), 1)
    Mc = (segr // 20 == comc).astype(jnp.float32)
    segc = lax.broadcasted_iota(jnp.int32, (16, NSEG), 1)
    comr = lax.broadcasted_iota(jnp.int32, (16, NSEG), 0)
    McT = (segc // 20 == comr).astype(jnp.float32)

    dn = (((1,), (0,)), ((), ()))
    hi = jax.lax.Precision.HIGHEST
    dot = functools.partial(lax.dot_general, dimension_numbers=dn,
                            precision=hi, preferred_element_type=jnp.float32)
    n_combo = dot(McT, present)
    safe_n = jnp.maximum(n_combo, 1.0)
    cnt_combo = dot(McT, cnt)
    anyp = cnt_combo > 0.0
    n_seg = dot(Mc, n_combo)
    wn = jnp.where(present > 0.0,
                   1.0 / (safe_cnt * jnp.maximum(n_seg, 1.0)), 0.0)

    m2 = m64 * m64
    sqn = jnp.sum(m2, axis=1, keepdims=True)
    sqn_row = lax.dot_general(jnp.ones((1, 64), jnp.float32), m2,
                              dimension_numbers=(((1,), (1,)), ((), ())),
                              precision=hi,
                              preferred_element_type=jnp.float32)
    regseg = jnp.sqrt(sqn + 1e-12) * present
    reg_combo = dot(McT, regseg) / safe_n

    G = lax.dot_general(m64, m64,
                        dimension_numbers=(((1,), (1,)), ((), ())),
                        precision=hi,
                        preferred_element_type=jnp.float32)
    pair_sq = jnp.maximum(sqn + sqn_row - 2.0 * G, 0.0)
    d = jnp.sqrt(pair_sq + 1e-12)
    h = jnp.maximum(3.0 - d, 0.0) ** 2
    ri = lax.broadcasted_iota(jnp.int32, (NSEG, NSEG), 0)
    ci = lax.broadcasted_iota(jnp.int32, (NSEG, NSEG), 1)
    same = (ri // 20) == (ci // 20)
    offdiag = ri != ci
    pres_row = lax.dot_general(jnp.ones((1, 1), jnp.float32), present,
                               dimension_numbers=(((1,), (1,)), ((), ())),
                               precision=hi,
                               preferred_element_type=jnp.float32)
    prespair = (present > 0.0) & (pres_row > 0.0)
    hm = jnp.where(same & offdiag & prespair, h, 0.0)
    dist_row = jnp.sum(hm, axis=1, keepdims=True)
    dist_combo = dot(McT, dist_row)
    denom = jnp.maximum((n_combo - 1.0) * n_combo, 1.0)
    dist_loss = jnp.where(n_combo < 2.0, 0.0, dist_combo / denom)
    distreg = jnp.sum(jnp.where(anyp, dist_loss + 0.001 * reg_combo, 0.0))

    batc = lax.broadcasted_iota(jnp.int32, (4, NSEG), 1)
    batr = lax.broadcasted_iota(jnp.int32, (4, NSEG), 0)
    MbT = (batc // 80 == batr).astype(jnp.float32)
    nb = dot(MbT, cnt)
    nbatch = jnp.sum((nb > 0.0).astype(jnp.float32))

    table_ref[...] = jnp.concatenate(
        [m64, wn, jnp.zeros((NSEG, W - 65), jnp.float32)], axis=1)
    svec_ref[...] = jnp.concatenate(
        [jnp.reshape(distreg, (1, 1)), jnp.reshape(nbatch, (1, 1)),
         jnp.zeros((1, 126), jnp.float32)], axis=1)


def _combine(partial):
    return pl.pallas_call(
        _combine_body,
        out_shape=(
            jax.ShapeDtypeStruct((NSEG, W), jnp.float32),
            jax.ShapeDtypeStruct((1, 128), jnp.float32),
        ),
    )(partial)


@functools.partial(
    pl.kernel,
    out_type=jax.ShapeDtypeStruct((NW * 16,), jnp.float32),
    mesh=_mesh,
    compiler_params=pltpu.CompilerParams(needs_layout_passes=False),
    scratch_types=[
        pltpu.VMEM((2, D * (CHUNK // 2) // 128, 128), jnp.float32),
        pltpu.VMEM((NSEG * W,), jnp.float32),
        pltpu.VMEM((8, 128), jnp.int32),
        pltpu.VMEM((16,), jnp.float32),
        pltpu.SemaphoreType.DMA,
    ],
)
def _var_points(xT, seg_in, table, cpart, xtbuf, tbuf, seg2, obuf, sem):
    cid = lax.axis_index("c")
    sid = lax.axis_index("s")
    wid = sid * NC + cid
    d1 = pltpu.async_copy(table, tbuf, sem)
    d2 = pltpu.async_copy(seg_in.at[pl.ds(wid * 8, 8)], seg2, sem)

    def inner(h, r, c, acc):
        segv = seg2[h * 4 + r, pl.ds(c * 16, 16)]
        fidx = segv * W
        sacc = jnp.zeros((16,), jnp.float32)
        for j in range(D):
            xj = xtbuf[h, j * 4 + r, pl.ds(c * 16, 16)]
            mj = plsc.load_gather(tbuf, [fidx])
            dv = xj - mj
            sacc = sacc + dv * dv
            fidx = fidx + 1
        wv = plsc.load_gather(tbuf, [fidx])
        s = sacc + 1e-12
        si = plsc.bitcast(s, jnp.int32)
        y = plsc.bitcast(lax.shift_right_logical(si, 1) + 0x1FBD1DF5,
                         jnp.float32)
        for _ in range(3):
            y = 0.5 * (y + s / y)
        hv = jnp.maximum(y - 0.5, 0.0)
        return acc + hv * hv * wv

    nrow = D * (CHUNK // 2) // 128
    varacc = jnp.zeros((16,), jnp.float32)
    rd0 = pltpu.async_copy(xT.at[pl.ds(wid * 2 * nrow, nrow)],
                           xtbuf.at[0], sem)
    rd0.wait()
    d1.wait()
    d2.wait()
    rd1 = pltpu.async_copy(xT.at[pl.ds((wid * 2 + 1) * nrow, nrow)],
                           xtbuf.at[1], sem)
    for h in range(2):
        if h == 1:
            rd1.wait()

        def outer(r, acc, h=h):
            return lax.fori_loop(0, 8, lambda c, a: inner(h, r, c, a), acc)

        varacc = lax.fori_loop(0, 4, outer, varacc)
    obuf[...] = varacc
    pltpu.sync_copy(obuf, cpart.at[pl.ds(wid * 16, 16)])


def _finish_body(cpart_ref, svec_ref, out_ref):
    sv = svec_ref[...]
    total = (jnp.sum(cpart_ref[...]) + sv[0, 0]) / sv[0, 1]
    out_ref[...] = jnp.reshape(total, (1, 1))


def _finish(cpart, svec):
    return pl.pallas_call(
        _finish_body,
        out_shape=jax.ShapeDtypeStruct((1, 1), jnp.float32),
    )(cpart, svec)


@jax.jit
def kernel(out, semantic_labels, group_labels):
    x = out.astype(jnp.float32)
    xpad = jnp.concatenate(
        [x, jnp.ones((N, 1), jnp.float32), jnp.zeros((N, W - D - 1), jnp.float32)],
        axis=1)
    bidx = semantic_labels[:, 3].astype(jnp.int32)
    slab = semantic_labels[:, 4].astype(jnp.int32)
    clab = group_labels[:, 4].astype(jnp.int32)
    zer = jnp.zeros((NSEG, W), jnp.float32)
    xT = x.T.reshape(D, NW * 2, CHUNK // 2).transpose(1, 0, 2).reshape(
        NW * 2 * D * (CHUNK // 2) // 128, 128)

    partial, seg = _seg_sums(xpad, bidx, slab, clab, zer)
    table, svec = _combine(partial)
    cpart = _var_points(xT, seg, table.reshape(NSEG * W))
    res = _finish(cpart.reshape(4, 128), svec)
    return res[0, 0]
